# baseline (speedup 1.0000x reference)
import jax
import jax.numpy as jnp
from jax import lax
from jax.experimental import pallas as pl
from jax.experimental.pallas import tpu as pltpu
from jax.experimental.pallas import tpu_sc as plsc

XK = 1000000
NBATCH = 16384
SMOOTH_EPS = 1e-6

NC = 2
NS = 16
NW = NC * NS
BPW = NBATCH // NW
ROWS_PER_W = BPW // 128


PAD_ROWS = 7936
PAD_N = PAD_ROWS * 128
NCHUNK = 4
CHUNK_ROWS = PAD_ROWS // NCHUNK


def _stats_body(main_hbm, out_ref, bufa, bufb, bufc, bufd, sema, semb, semc, semd):
    bufs = (bufa, bufb, bufc, bufd)
    sems = (sema, semb, semc, semd)

    def copy(k):
        src = main_hbm.at[pl.ds(k * CHUNK_ROWS, CHUNK_ROWS), :]
        return pltpu.make_async_copy(src, bufs[k], sems[k])

    for k in range(NCHUNK):
        copy(k).start()

    acc_m = None
    acc_s = None
    for k in range(NCHUNK):
        copy(k).wait()
        x = bufs[k][...]
        bm = jnp.max(x, axis=0, keepdims=True)
        if acc_m is None:
            acc_m = bm
            acc_s = jnp.sum(jnp.exp(x - bm), axis=0, keepdims=True)
        else:
            new_m = jnp.maximum(acc_m, bm)
            acc_s = acc_s * jnp.exp(acc_m - new_m) + jnp.sum(
                jnp.exp(x - new_m), axis=0, keepdims=True
            )
            acc_m = new_m

    m = jnp.max(acc_m)
    s = jnp.sum(acc_s * jnp.exp(acc_m - m))
    scale = (1.0 - SMOOTH_EPS) / s
    row = lax.broadcasted_iota(jnp.int32, (8, 128), 0)
    out_ref[...] = jnp.where(row < 1, m, scale)


def _gather_body(emb_hbm, idx_hbm, out_hbm, idx_v, rows_v, sem):
    wid = lax.axis_index("s") * NC + lax.axis_index("c")
    base = wid * ROWS_PER_W
    pltpu.sync_copy(idx_hbm.at[pl.ds(base, ROWS_PER_W)], idx_v)
    copies = [
        pltpu.async_copy(emb_hbm.at[idx_v.at[j]], rows_v.at[j], sem)
        for j in range(ROWS_PER_W)
    ]
    for c in copies:
        c.wait()
    pltpu.sync_copy(rows_v, out_hbm.at[pl.ds(base, ROWS_PER_W)])


def _apply_body(g_ref, stats_ref, out_ref):
    st = stats_ref[...]
    m = st[0, 0]
    scale = st[1, 0]
    g = g_ref[...]
    out_ref[...] = jnp.exp(g - m) * scale + jnp.float32(SMOOTH_EPS / XK)


@jax.jit
def kernel(inputs, embedding):
    idx = inputs.reshape(NBATCH // 128, 128).astype(jnp.int32)

    emb_pad = jnp.concatenate(
        [embedding, jnp.full((PAD_N - XK,), -jnp.inf, jnp.float32)]
    ).reshape(PAD_ROWS, 128)
    stats = pl.pallas_call(
        _stats_body,
        in_specs=[pl.BlockSpec(memory_space=pl.ANY)],
        out_shape=jax.ShapeDtypeStruct((8, 128), jnp.float32),
        scratch_shapes=[pltpu.VMEM((CHUNK_ROWS, 128), jnp.float32)] * 4
        + [pltpu.SemaphoreType.DMA] * 4,
    )(emb_pad)

    mesh = plsc.VectorSubcoreMesh(core_axis_name="c", subcore_axis_name="s")
    gathered = pl.kernel(
        _gather_body,
        mesh=mesh,
        out_type=jax.ShapeDtypeStruct((NBATCH // 128, 128), jnp.float32),
        scratch_types=[
            pltpu.VMEM((ROWS_PER_W, 128), jnp.int32),
            pltpu.VMEM((ROWS_PER_W, 128), jnp.float32),
            pltpu.SemaphoreType.DMA,
        ],
    )(embedding, idx)

    out = pl.pallas_call(
        _apply_body,
        out_shape=jax.ShapeDtypeStruct((NBATCH // 128, 128), jnp.float32),
    )(gathered, stats)

    return out.reshape(NBATCH, 1)


# speedup vs baseline: 1.0930x; 1.0930x over previous
import jax
import jax.numpy as jnp
from jax import lax
from jax.experimental import pallas as pl
from jax.experimental.pallas import tpu as pltpu
from jax.experimental.pallas import tpu_sc as plsc

XK = 1000000
NBATCH = 16384
SMOOTH_EPS = 1e-6

NC = 1
NS = 16
NW = NC * NS
BPW = NBATCH // NW
ROWS_PER_W = BPW // 128


PAD_ROWS = 7936
PAD_N = PAD_ROWS * 128
NCHUNK = 16
CHUNK_ROWS = PAD_ROWS // NCHUNK


def _stats_body(main_ref, out_ref):
    acc_m = None
    acc_s = None
    for k in range(NCHUNK):
        x = main_ref[pl.ds(k * CHUNK_ROWS, CHUNK_ROWS), :]
        bm = jnp.max(x, axis=0, keepdims=True)
        if acc_m is None:
            acc_m = bm
            acc_s = jnp.sum(jnp.exp(x - bm), axis=0, keepdims=True)
        else:
            new_m = jnp.maximum(acc_m, bm)
            acc_s = acc_s * jnp.exp(acc_m - new_m) + jnp.sum(
                jnp.exp(x - new_m), axis=0, keepdims=True
            )
            acc_m = new_m

    m = jnp.max(acc_m)
    s = jnp.sum(acc_s * jnp.exp(acc_m - m))
    scale = (1.0 - SMOOTH_EPS) / s
    row = lax.broadcasted_iota(jnp.int32, (8, 128), 0)
    out_ref[...] = jnp.where(row < 1, m, scale)


def _gather_body(emb_hbm, idx_hbm, out_hbm, idx_v, rows_v, sem):
    wid = lax.axis_index("s") * NC + lax.axis_index("c")
    base = wid * ROWS_PER_W
    pltpu.sync_copy(idx_hbm.at[pl.ds(base, ROWS_PER_W)], idx_v)
    copies = [
        pltpu.async_copy(emb_hbm.at[idx_v.at[j]], rows_v.at[j], sem)
        for j in range(ROWS_PER_W)
    ]
    for c in copies:
        c.wait()
    pltpu.sync_copy(rows_v, out_hbm.at[pl.ds(base, ROWS_PER_W)])


def _apply_body(g_ref, stats_ref, out_ref):
    st = stats_ref[...]
    m = st[0, 0]
    scale = st[1, 0]
    g = g_ref[...]
    out_ref[...] = jnp.exp(g - m) * scale + jnp.float32(SMOOTH_EPS / XK)


@jax.jit
def kernel(inputs, embedding):
    idx = inputs.reshape(NBATCH // 128, 128).astype(jnp.int32)

    emb_pad = jnp.concatenate(
        [embedding, jnp.full((PAD_N - XK,), -jnp.inf, jnp.float32)]
    ).reshape(PAD_ROWS, 128)
    stats = pl.pallas_call(
        _stats_body,
        out_shape=jax.ShapeDtypeStruct((8, 128), jnp.float32),
    )(emb_pad)

    mesh = plsc.VectorSubcoreMesh(core_axis_name="c", subcore_axis_name="s", num_cores=1)
    gathered = pl.kernel(
        _gather_body,
        mesh=mesh,
        out_type=jax.ShapeDtypeStruct((NBATCH // 128, 128), jnp.float32),
        scratch_types=[
            pltpu.VMEM((ROWS_PER_W, 128), jnp.int32),
            pltpu.VMEM((ROWS_PER_W, 128), jnp.float32),
            pltpu.SemaphoreType.DMA,
        ],
    )(embedding, idx)

    out = pl.pallas_call(
        _apply_body,
        out_shape=jax.ShapeDtypeStruct((NBATCH // 128, 128), jnp.float32),
    )(gathered, stats)

    return out.reshape(NBATCH, 1)
